# baseline (speedup 1.0000x reference)
import functools

import jax
import jax.numpy as jnp
from jax import lax
from jax.experimental import pallas as pl
from jax.experimental.pallas import tpu as pltpu
from jax.experimental.pallas import tpu_sc as plsc

_CHUNK = 128


def _sc_gather(table, idx3):
    nw, nchunk, _ = idx3.shape

    mesh = plsc.VectorSubcoreMesh(
        core_axis_name="c", subcore_axis_name="s", num_cores=1
    )

    @functools.partial(
        pl.kernel,
        mesh=mesh,
        out_type=jax.ShapeDtypeStruct((nw, nchunk, _CHUNK), jnp.float32),
        compiler_params=pltpu.CompilerParams(needs_layout_passes=False),
        scratch_types=[
            pltpu.VMEM((nchunk, _CHUNK), jnp.int32),
            pltpu.VMEM((nchunk, _CHUNK), jnp.float32),
            pltpu.SemaphoreType.DMA,
            pltpu.SemaphoreType.DMA,
            pltpu.SemaphoreType.DMA,
        ],
    )
    def k(table_hbm, idx_hbm, out_hbm, idx_v, out_v, sem_i, sem_g, sem_o):
        wid = lax.axis_index("s")
        pltpu.async_copy(idx_hbm.at[wid], idx_v, sem_i).wait()
        gathers = [
            pltpu.async_copy(
                table_hbm.at[idx_v.at[j]], out_v.at[j], sem_g)
            for j in range(nchunk)
        ]
        for g in gathers:
            g.wait()
        pltpu.async_copy(out_v, out_hbm.at[wid], sem_o).wait()

    return k(table, idx3)


def kernel(inData, inIndex, inShape):
    nbatch = inIndex.shape[0]
    info = plsc.get_sparse_core_info()
    nw = info.num_subcores
    idx3 = inIndex.astype(jnp.int32).reshape(nw, -1, _CHUNK)
    out = _sc_gather(inData.astype(jnp.float32), idx3)
    return out.reshape((nbatch,) + (1,) * (len(inShape) - 1))


# speedup vs baseline: 1.5122x; 1.5122x over previous
import functools

import jax
import jax.numpy as jnp
from jax import lax
from jax.experimental import pallas as pl
from jax.experimental.pallas import tpu as pltpu
from jax.experimental.pallas import tpu_sc as plsc

_LANES = 16


def _sc_gather(table, idx):
    B = idx.shape[0]
    T = table.shape[0]
    info = plsc.get_sparse_core_info()
    nc, ns = 1, info.num_subcores
    nw = nc * ns
    b_per_w = B // nw

    mesh = plsc.VectorSubcoreMesh(
        core_axis_name="c", subcore_axis_name="s", num_cores=1
    )

    @functools.partial(
        pl.kernel,
        mesh=mesh,
        out_type=jax.ShapeDtypeStruct((B,), jnp.float32),
        compiler_params=pltpu.CompilerParams(needs_layout_passes=False),
        scratch_types=[
            pltpu.VMEM((T,), jnp.float32),
            pltpu.VMEM((b_per_w,), jnp.int32),
            pltpu.VMEM((b_per_w,), jnp.float32),
            pltpu.SemaphoreType.DMA,
            pltpu.SemaphoreType.DMA,
            pltpu.SemaphoreType.DMA,
            pltpu.SemaphoreType.DMA,
        ],
    )
    def k(table_hbm, idx_hbm, out_hbm, table_v, idx_v, out_v,
          sem_t, sem_i0, sem_i1, sem_o):
        wid = lax.axis_index("s") * nc + lax.axis_index("c")
        base = wid * b_per_w
        half = b_per_w // 2
        cp_t = pltpu.async_copy(table_hbm, table_v, sem_t)
        cp_i0 = pltpu.async_copy(
            idx_hbm.at[pl.ds(base, half)], idx_v.at[pl.ds(0, half)], sem_i0)
        cp_i1 = pltpu.async_copy(
            idx_hbm.at[pl.ds(base + half, half)],
            idx_v.at[pl.ds(half, half)], sem_i1)
        cp_i0.wait()
        cp_t.wait()

        @plsc.parallel_loop(0, half, step=_LANES, unroll=8)
        def _gather0(i):
            ids = idx_v[pl.ds(i, _LANES)]
            out_v[pl.ds(i, _LANES)] = plsc.load_gather(table_v, [ids])

        cp_o0 = pltpu.async_copy(
            out_v.at[pl.ds(0, half)], out_hbm.at[pl.ds(base, half)], sem_o)
        cp_i1.wait()

        @plsc.parallel_loop(half, b_per_w, step=_LANES, unroll=8)
        def _gather1(i):
            ids = idx_v[pl.ds(i, _LANES)]
            out_v[pl.ds(i, _LANES)] = plsc.load_gather(table_v, [ids])

        cp_o1 = pltpu.async_copy(
            out_v.at[pl.ds(half, half)],
            out_hbm.at[pl.ds(base + half, half)], sem_o)
        cp_o0.wait()
        cp_o1.wait()

    return k(table, idx)


def kernel(inData, inIndex, inShape):
    nbatch = inIndex.shape[0]
    out = _sc_gather(inData.astype(jnp.float32), inIndex.astype(jnp.int32))
    return out.reshape((nbatch,) + (1,) * (len(inShape) - 1))


# speedup vs baseline: 1.5242x; 1.0080x over previous
import functools

import jax
import jax.numpy as jnp
from jax import lax
from jax.experimental import pallas as pl
from jax.experimental.pallas import tpu as pltpu
from jax.experimental.pallas import tpu_sc as plsc

_LANES = 16


def _sc_gather(table, idx):
    B = idx.shape[0]
    T = table.shape[0]
    nc, ns = 1, 8
    nw = nc * ns
    b_per_w = B // nw

    mesh = plsc.VectorSubcoreMesh(
        core_axis_name="c", subcore_axis_name="s", num_cores=1, num_subcores=ns
    )

    @functools.partial(
        pl.kernel,
        mesh=mesh,
        out_type=jax.ShapeDtypeStruct((B,), jnp.float32),
        compiler_params=pltpu.CompilerParams(needs_layout_passes=False),
        scratch_types=[
            pltpu.VMEM((T,), jnp.float32),
            pltpu.VMEM((b_per_w,), jnp.int32),
            pltpu.VMEM((b_per_w,), jnp.float32),
            pltpu.SemaphoreType.DMA,
            pltpu.SemaphoreType.DMA,
            pltpu.SemaphoreType.DMA,
            pltpu.SemaphoreType.DMA,
        ],
    )
    def k(table_hbm, idx_hbm, out_hbm, table_v, idx_v, out_v,
          sem_t, sem_i0, sem_i1, sem_o):
        wid = lax.axis_index("s") * nc + lax.axis_index("c")
        base = wid * b_per_w
        half = b_per_w // 2
        cp_t = pltpu.async_copy(table_hbm, table_v, sem_t)
        cp_i0 = pltpu.async_copy(
            idx_hbm.at[pl.ds(base, half)], idx_v.at[pl.ds(0, half)], sem_i0)
        cp_i1 = pltpu.async_copy(
            idx_hbm.at[pl.ds(base + half, half)],
            idx_v.at[pl.ds(half, half)], sem_i1)
        cp_i0.wait()
        cp_t.wait()

        @plsc.parallel_loop(0, half, step=_LANES, unroll=8)
        def _gather0(i):
            ids = idx_v[pl.ds(i, _LANES)]
            out_v[pl.ds(i, _LANES)] = plsc.load_gather(table_v, [ids])

        cp_o0 = pltpu.async_copy(
            out_v.at[pl.ds(0, half)], out_hbm.at[pl.ds(base, half)], sem_o)
        cp_i1.wait()

        @plsc.parallel_loop(half, b_per_w, step=_LANES, unroll=8)
        def _gather1(i):
            ids = idx_v[pl.ds(i, _LANES)]
            out_v[pl.ds(i, _LANES)] = plsc.load_gather(table_v, [ids])

        cp_o1 = pltpu.async_copy(
            out_v.at[pl.ds(half, half)],
            out_hbm.at[pl.ds(base + half, half)], sem_o)
        cp_o0.wait()
        cp_o1.wait()

    return k(table, idx)


def kernel(inData, inIndex, inShape):
    nbatch = inIndex.shape[0]
    out = _sc_gather(inData.astype(jnp.float32), inIndex.astype(jnp.int32))
    return out.reshape((nbatch,) + (1,) * (len(inShape) - 1))
